# baseline (speedup 1.0000x reference)
import functools

import jax
import jax.numpy as jnp
from jax import lax
from jax.experimental import pallas as pl
from jax.experimental.pallas import tpu as pltpu
from jax.experimental.pallas import tpu_sc as plsc

N = 50000
E = 800000
HID = 32
OUT = 4
NUM_GRAPHS = 128

NTILES = 32
CH = 128
NCH = 196
EPT = NCH * CH
EPAD = NTILES * EPT
KC = 28
NST = NCH // KC
ACC_ROWS = 50048
ZROWS = ACC_ROWS // 16

BN = 1000
GRID = N // BN

_mesh = plsc.VectorSubcoreMesh(core_axis_name="c", subcore_axis_name="s")
_sc_params = pltpu.CompilerParams(use_tc_tiling_on_sc=False)


@functools.partial(
    pl.kernel, mesh=_mesh,
    out_type=jax.ShapeDtypeStruct((2, ACC_ROWS, 16), jnp.float32),
    scratch_types=[
        pltpu.VMEM((KC, CH), jnp.int32),
        pltpu.VMEM((CH, 16), jnp.float32),
        pltpu.VMEM_SHARED((ACC_ROWS, 16), jnp.float32),
        pltpu.SemaphoreType.DMA,
    ],
    compiler_params=_sc_params,
)
def _deg_kernel(dst_hbm, zeros_hbm, ones_hbm, out_hbm, dstv, onesv, deg, sems):
    c = lax.axis_index("c")
    s = lax.axis_index("s")
    wid = c * 16 + s
    pltpu.sync_copy(ones_hbm, onesv)
    pltpu.sync_copy(zeros_hbm, deg.at[pl.ds(s * ZROWS, ZROWS)])
    plsc.subcore_barrier()

    def outer(t, carry):
        pltpu.sync_copy(dst_hbm.at[wid, pl.ds(t * KC, KC)], dstv)

        def fire(j, cc):
            pltpu.async_copy(onesv, deg.at[dstv.at[j]], sems, add=True)
            return cc

        lax.fori_loop(0, KC, fire, carry)

        def drain(j, cc):
            pltpu.make_async_copy(onesv, deg.at[dstv.at[j]], sems).wait()
            return cc

        return lax.fori_loop(0, KC, drain, carry)

    lax.fori_loop(0, NST, outer, 0)
    plsc.subcore_barrier()
    pltpu.sync_copy(deg.at[pl.ds(s * ZROWS, ZROWS)],
                    out_hbm.at[c, pl.ds(s * ZROWS, ZROWS)])


@functools.partial(
    pl.kernel, mesh=_mesh,
    out_type=jax.ShapeDtypeStruct((2, ACC_ROWS, HID), jnp.float32),
    scratch_types=[
        pltpu.VMEM((KC, CH), jnp.int32),
        pltpu.VMEM((KC, CH), jnp.int32),
        pltpu.VMEM((CH, HID), jnp.float32),
        pltpu.VMEM((CH, HID), jnp.float32),
        pltpu.VMEM_SHARED((ACC_ROWS, HID), jnp.float32),
        pltpu.SemaphoreType.DMA,
        pltpu.SemaphoreType.DMA,
    ],
    compiler_params=_sc_params,
)
def _edge_kernel(g_hbm, src_hbm, dst_hbm, zeros_hbm, out_hbm,
                 srcv, dstv, rows0, rows1, acc, semg0, semg1):
    c = lax.axis_index("c")
    s = lax.axis_index("s")
    wid = c * 16 + s
    pltpu.sync_copy(zeros_hbm, acc.at[pl.ds(s * ZROWS, ZROWS)])
    plsc.subcore_barrier()

    def outer(t, carry):
        pltpu.sync_copy(src_hbm.at[wid, pl.ds(t * KC, KC)], srcv)
        pltpu.sync_copy(dst_hbm.at[wid, pl.ds(t * KC, KC)], dstv)
        pltpu.async_copy(g_hbm.at[srcv.at[0]], rows0, semg0)

        def body(jj, cc):
            j0 = 2 * jj
            j1 = j0 + 1
            pltpu.async_copy(g_hbm.at[srcv.at[j1]], rows1, semg1)
            pltpu.make_async_copy(g_hbm.at[srcv.at[j0]], rows0, semg0).wait()
            pltpu.sync_copy(rows0, acc.at[dstv.at[j0]], add=True)

            @pl.when(jj < KC // 2 - 1)
            def _():
                pltpu.async_copy(g_hbm.at[srcv.at[j0 + 2]], rows0, semg0)

            pltpu.make_async_copy(g_hbm.at[srcv.at[j1]], rows1, semg1).wait()
            pltpu.sync_copy(rows1, acc.at[dstv.at[j1]], add=True)
            return cc

        return lax.fori_loop(0, KC // 2, body, carry)

    lax.fori_loop(0, NST, outer, 0)
    plsc.subcore_barrier()
    pltpu.sync_copy(acc.at[pl.ds(s * ZROWS, ZROWS)],
                    out_hbm.at[c, pl.ds(s * ZROWS, ZROWS)])


def _dinv_from(degp):
    d = degp[0, :, 0:1] + degp[1, :, 0:1] + 1.0
    return lax.rsqrt(d)


def _tc1_body(cx, bx, px, sx, Wc, bc, Wb, bb, Wp, bp, Ws, bs, Wg1,
              h1_out):
    ch = jnp.maximum(jnp.dot(cx[...], Wc[...],
                             preferred_element_type=jnp.float32) + bc[...], 0.0)
    bh = jnp.maximum(jnp.dot(bx[...], Wb[...],
                             preferred_element_type=jnp.float32) + bb[...], 0.0)
    ph = jnp.maximum(jnp.dot(px[...], Wp[...],
                             preferred_element_type=jnp.float32) + bp[...], 0.0)
    sh = jnp.maximum(jnp.dot(sx[...], Ws[...],
                             preferred_element_type=jnp.float32) + bs[...], 0.0)
    x0 = jnp.concatenate([ch, bh, ph, sh], axis=1)
    h1_out[...] = jnp.dot(x0, Wg1[...], preferred_element_type=jnp.float32)


def _scale_body(h, degp, g_out):
    g_out[...] = h[...] * _dinv_from(degp[...])


def _tc_mid_body(accp, g, degp, b, W, g_out):
    dinv = _dinv_from(degp[...])
    acc = accp[0] + accp[1]
    x = jnp.maximum(dinv * (acc + g[...]) + b[...], 0.0)
    g_out[...] = jnp.dot(x, W[...], preferred_element_type=jnp.float32) * dinv


def _tc3_body(accp, g, degp, batchr, b2, Wl1, bl1, Wl2, bl2, out,
              sums, counts):
    i = pl.program_id(0)

    @pl.when(i == 0)
    def _():
        sums[...] = jnp.zeros_like(sums)
        counts[...] = jnp.zeros_like(counts)

    dinv = _dinv_from(degp[...])
    acc = accp[0] + accp[1]
    x2 = jnp.maximum(dinv * (acc + g[...]) + b2[...], 0.0)
    gids = batchr[0]
    onehot = (lax.broadcasted_iota(jnp.int32, (NUM_GRAPHS, BN), 0)
              == gids).astype(jnp.float32)
    sums[...] += jnp.dot(onehot, x2, preferred_element_type=jnp.float32)
    counts[...] += jnp.sum(onehot, axis=1, keepdims=True)

    @pl.when(i == GRID - 1)
    def _():
        pooled = sums[...] / jnp.maximum(counts[:, 0:1], 1.0)
        h = jnp.maximum(jnp.dot(pooled, Wl1[...],
                                preferred_element_type=jnp.float32) + bl1[...],
                        0.0)
        logits = jnp.dot(h, Wl2[...],
                         preferred_element_type=jnp.float32) + bl2[...]
        m = jnp.max(logits, axis=-1, keepdims=True)
        sh = logits - m
        out[...] = sh - jnp.log(jnp.sum(jnp.exp(sh), axis=-1, keepdims=True))


def _row_spec(w):
    return pl.BlockSpec((BN, w), lambda i: (i, 0))


def _full_spec(shape):
    return pl.BlockSpec(shape, lambda i: tuple(0 for _ in shape))


_degp_spec = pl.BlockSpec((2, BN, 16), lambda i: (0, i, 0))
_accp_spec = pl.BlockSpec((2, BN, HID), lambda i: (0, i, 0))


def _tc1_call(cx, bx, px, sx, Wc, bc, Wb, bb, Wp, bp, Ws, bs, Wg1):
    return pl.pallas_call(
        _tc1_body,
        grid=(GRID,),
        in_specs=[_row_spec(310), _row_spec(768), _row_spec(10), _row_spec(300),
                  _full_spec((310, HID)), _full_spec((1, HID)),
                  _full_spec((768, HID)), _full_spec((1, HID)),
                  _full_spec((10, HID)), _full_spec((1, HID)),
                  _full_spec((300, HID)), _full_spec((1, HID)),
                  _full_spec((4 * HID, HID))],
        out_specs=_row_spec(HID),
        out_shape=jax.ShapeDtypeStruct((N, HID), jnp.float32),
    )(cx, bx, px, sx, Wc, bc, Wb, bb, Wp, bp, Ws, bs, Wg1)


def _scale_call(h, degp):
    return pl.pallas_call(
        _scale_body,
        grid=(GRID,),
        in_specs=[_row_spec(HID), _degp_spec],
        out_specs=_row_spec(HID),
        out_shape=jax.ShapeDtypeStruct((N, HID), jnp.float32),
    )(h, degp)


def _tc_mid_call(accp, g, degp, b, W):
    return pl.pallas_call(
        _tc_mid_body,
        grid=(GRID,),
        in_specs=[_accp_spec, _row_spec(HID), _degp_spec,
                  _full_spec((1, HID)), _full_spec((HID, HID))],
        out_specs=_row_spec(HID),
        out_shape=jax.ShapeDtypeStruct((N, HID), jnp.float32),
    )(accp, g, degp, b, W)


def _tc3_call(accp, g, degp, batchr, b2, Wl1, bl1, Wl2, bl2):
    return pl.pallas_call(
        _tc3_body,
        grid=(GRID,),
        in_specs=[_accp_spec, _row_spec(HID), _degp_spec,
                  pl.BlockSpec((1, 1, BN), lambda i: (i, 0, 0)),
                  _full_spec((1, HID)), _full_spec((HID, HID)),
                  _full_spec((1, HID)), _full_spec((HID, OUT)),
                  _full_spec((1, OUT))],
        out_specs=_full_spec((NUM_GRAPHS, OUT)),
        out_shape=jax.ShapeDtypeStruct((NUM_GRAPHS, OUT), jnp.float32),
        scratch_shapes=[pltpu.VMEM((NUM_GRAPHS, HID), jnp.float32),
                        pltpu.VMEM((NUM_GRAPHS, NUM_GRAPHS), jnp.float32)],
    )(accp, g, degp, batchr, b2, Wl1, bl1, Wl2, bl2)


def kernel(content_x, bert_x, profile_x, spacy_x, edge_index, batch,
           Wc, bc, Wb, bb, Wp, bp, Ws, bs,
           Wg1, bg1, Wg2, bg2, Wl1, bl1, Wl2, bl2):
    src, dst = edge_index[0], edge_index[1]
    pad = EPAD - E
    srcp = jnp.concatenate([src, jnp.zeros((pad,), jnp.int32)]
                           ).reshape(NTILES, NCH, CH)
    dstp = jnp.concatenate([dst, jnp.full((pad,), N, jnp.int32)]
                           ).reshape(NTILES, NCH, CH)
    zeros16 = jnp.zeros((ZROWS, 16), jnp.float32)
    zeros32 = jnp.zeros((ZROWS, HID), jnp.float32)
    ones16 = jnp.ones((CH, 16), jnp.float32)
    b1 = bc.reshape(1, HID)
    b2 = bb.reshape(1, HID)
    b3 = bp.reshape(1, HID)
    b4 = bs.reshape(1, HID)

    degp = _deg_kernel(dstp, zeros16, ones16)
    h1 = _tc1_call(content_x, bert_x, profile_x, spacy_x,
                   Wc, b1, Wb, b2, Wp, b3, Ws, b4, Wg1)
    g1 = _scale_call(h1, degp)
    acc1 = _edge_kernel(g1, srcp, dstp, zeros32)
    g2 = _tc_mid_call(acc1, g1, degp, bg1.reshape(1, HID), Wg2)
    acc2 = _edge_kernel(g2, srcp, dstp, zeros32)
    return _tc3_call(acc2, g2, degp, batch.reshape(GRID, 1, BN),
                     bg2.reshape(1, HID), Wl1, bl1.reshape(1, HID),
                     Wl2, bl2.reshape(1, OUT))


# speedup vs baseline: 1.2841x; 1.2841x over previous
import functools

import jax
import jax.numpy as jnp
from jax import lax
from jax.experimental import pallas as pl
from jax.experimental.pallas import tpu as pltpu
from jax.experimental.pallas import tpu_sc as plsc

N = 50000
E = 800000
HID = 32
OUT = 4
NUM_GRAPHS = 128

NTILES = 32
CH = 128
NCH = 196
EPT = NCH * CH
EPAD = NTILES * EPT
KC = 28
NST = NCH // KC
PK = 4
BN = 1024
BP = BN // PK
GRID = -(-N // BN)
NBPAD = GRID * BN

ACC_ROWS = 50304
ZROWS = ACC_ROWS // 16
NPROWS = ACC_ROWS // PK

_mesh = plsc.VectorSubcoreMesh(core_axis_name="c", subcore_axis_name="s")
_sc_params = pltpu.CompilerParams(use_tc_tiling_on_sc=False)


@functools.partial(
    pl.kernel, mesh=_mesh,
    out_type=jax.ShapeDtypeStruct((2, ACC_ROWS, HID), jnp.float32),
    scratch_types=[
        pltpu.VMEM((KC, CH), jnp.int32),
        pltpu.VMEM((CH, HID), jnp.float32),
        pltpu.VMEM_SHARED((ACC_ROWS, HID), jnp.float32),
        pltpu.SemaphoreType.DMA,
    ],
    compiler_params=_sc_params,
)
def _deg_kernel(srcdst_hbm, zeros_hbm, ones_hbm, out_hbm, dstv, onesv, deg,
                sems):
    c = lax.axis_index("c")
    s = lax.axis_index("s")
    wid = c * 16 + s
    pltpu.sync_copy(ones_hbm, onesv)
    pltpu.sync_copy(zeros_hbm, deg.at[pl.ds(s * ZROWS, ZROWS)])
    plsc.subcore_barrier()

    def outer(t, carry):
        pltpu.sync_copy(srcdst_hbm.at[1, wid, pl.ds(t * KC, KC)], dstv)

        def fire(j, cc):
            pltpu.async_copy(onesv, deg.at[dstv.at[j]], sems, add=True)
            return cc

        lax.fori_loop(0, KC, fire, carry)

        def drain(j, cc):
            pltpu.make_async_copy(onesv, deg.at[dstv.at[j]], sems).wait()
            return cc

        return lax.fori_loop(0, KC, drain, carry)

    lax.fori_loop(0, NST, outer, 0)
    plsc.subcore_barrier()
    pltpu.sync_copy(deg.at[pl.ds(s * ZROWS, ZROWS)],
                    out_hbm.at[c, pl.ds(s * ZROWS, ZROWS)])


@functools.partial(
    pl.kernel, mesh=_mesh,
    out_type=jax.ShapeDtypeStruct((2, ACC_ROWS, HID), jnp.float32),
    scratch_types=[
        pltpu.VMEM((KC, CH), jnp.int32),
        pltpu.VMEM((KC, CH), jnp.int32),
        pltpu.VMEM((CH, HID), jnp.float32),
        pltpu.VMEM((CH, HID), jnp.float32),
        pltpu.VMEM_SHARED((ACC_ROWS, HID), jnp.float32),
        pltpu.SemaphoreType.DMA,
        pltpu.SemaphoreType.DMA,
    ],
    compiler_params=_sc_params,
)
def _edge_kernel(g_hbm, srcdst_hbm, zeros_hbm, out_hbm,
                 srcv, dstv, rows0, rows1, acc, semg0, semg1):
    c = lax.axis_index("c")
    s = lax.axis_index("s")
    wid = c * 16 + s
    pltpu.sync_copy(zeros_hbm, acc.at[pl.ds(s * ZROWS, ZROWS)])
    plsc.subcore_barrier()

    def outer(t, carry):
        pltpu.sync_copy(srcdst_hbm.at[0, wid, pl.ds(t * KC, KC)], srcv)
        pltpu.sync_copy(srcdst_hbm.at[1, wid, pl.ds(t * KC, KC)], dstv)
        pltpu.async_copy(g_hbm.at[srcv.at[0]], rows0, semg0)

        def body(jj, cc):
            j0 = 2 * jj
            j1 = j0 + 1
            pltpu.async_copy(g_hbm.at[srcv.at[j1]], rows1, semg1)
            pltpu.make_async_copy(g_hbm.at[srcv.at[j0]], rows0, semg0).wait()
            pltpu.sync_copy(rows0, acc.at[dstv.at[j0]], add=True)

            @pl.when(jj < KC // 2 - 1)
            def _():
                pltpu.async_copy(g_hbm.at[srcv.at[j0 + 2]], rows0, semg0)

            pltpu.make_async_copy(g_hbm.at[srcv.at[j1]], rows1, semg1).wait()
            pltpu.sync_copy(rows1, acc.at[dstv.at[j1]], add=True)
            return cc

        return lax.fori_loop(0, KC // 2, body, carry)

    lax.fori_loop(0, NST, outer, 0)
    plsc.subcore_barrier()
    pltpu.sync_copy(acc.at[pl.ds(s * ZROWS, ZROWS)],
                    out_hbm.at[c, pl.ds(s * ZROWS, ZROWS)])


def _dinvp_from(degp):
    return lax.rsqrt(degp[0] + degp[1] + 1.0)


def _tc1_body(cx, bx, px, sx, degp, Wc, bc, Wb, bb, Wp, bp, Ws, bs, Wg1,
              g1_out):
    ch = jnp.maximum(jnp.dot(cx[...], Wc[...],
                             preferred_element_type=jnp.float32) + bc[...], 0.0)
    bh = jnp.maximum(jnp.dot(bx[...], Wb[...],
                             preferred_element_type=jnp.float32) + bb[...], 0.0)
    ph = jnp.maximum(jnp.dot(px[...], Wp[...],
                             preferred_element_type=jnp.float32) + bp[...], 0.0)
    sh = jnp.maximum(jnp.dot(sx[...], Ws[...],
                             preferred_element_type=jnp.float32) + bs[...], 0.0)
    x0 = jnp.concatenate([ch, bh, ph, sh], axis=1)
    h1 = jnp.dot(x0, Wg1[...], preferred_element_type=jnp.float32)
    h1p = jnp.concatenate([h1[k * BP:(k + 1) * BP] for k in range(PK)],
                          axis=1)
    g1_out[...] = h1p * _dinvp_from(degp[...])


def _tc2_body(accp, g, degp, b4, W4, g_out):
    dinvp = _dinvp_from(degp[...])
    acc = accp[0] + accp[1]
    x = jnp.maximum(dinvp * (acc + g[...]) + b4[...], 0.0)
    g_out[...] = jnp.dot(x, W4[...], preferred_element_type=jnp.float32) * dinvp


def _tc3_body(accp, g, degp, batchq, b4, Wl1, bl1, Wl2, bl2, out,
              sums, counts):
    i = pl.program_id(0)

    @pl.when(i == 0)
    def _():
        sums[...] = jnp.zeros_like(sums)
        counts[...] = jnp.zeros_like(counts)

    dinvp = _dinvp_from(degp[...])
    acc = accp[0] + accp[1]
    x2 = jnp.maximum(dinvp * (acc + g[...]) + b4[...], 0.0)
    node = (i * BN + lax.broadcasted_iota(jnp.int32, (BP, PK * HID), 0)
            + BP * (lax.broadcasted_iota(jnp.int32, (BP, PK * HID), 1) // HID))
    x2 = jnp.where(node < N, x2, 0.0)
    giota = lax.broadcasted_iota(jnp.int32, (NUM_GRAPHS, BP), 0)
    for q in range(PK):
        onehot = (giota == batchq[0, q]).astype(jnp.float32)
        sums[...] += jnp.dot(onehot, x2[:, q * HID:(q + 1) * HID],
                             preferred_element_type=jnp.float32)
        counts[...] += jnp.sum(onehot, axis=1, keepdims=True)

    @pl.when(i == GRID - 1)
    def _():
        pooled = sums[...] / jnp.maximum(counts[:, 0:1], 1.0)
        h = jnp.maximum(jnp.dot(pooled, Wl1[...],
                                preferred_element_type=jnp.float32) + bl1[...],
                        0.0)
        logits = jnp.dot(h, Wl2[...],
                         preferred_element_type=jnp.float32) + bl2[...]
        m = jnp.max(logits, axis=-1, keepdims=True)
        sh = logits - m
        out[...] = sh - jnp.log(jnp.sum(jnp.exp(sh), axis=-1, keepdims=True))


def _row_spec(w):
    return pl.BlockSpec((BN, w), lambda i: (i, 0))


def _full_spec(shape):
    return pl.BlockSpec(shape, lambda i: tuple(0 for _ in shape))


_pk_spec = pl.BlockSpec((BP, PK * HID), lambda i: (i, 0))
_pk2_spec = pl.BlockSpec((2, BP, PK * HID), lambda i: (0, i, 0))


def _tc1_call(cx, bx, px, sx, degp, Wc, bc, Wb, bb, Wp, bp, Ws, bs, Wg1):
    return pl.pallas_call(
        _tc1_body,
        grid=(GRID,),
        in_specs=[_row_spec(310), _row_spec(768), _row_spec(10), _row_spec(300),
                  _pk2_spec,
                  _full_spec((310, HID)), _full_spec((1, HID)),
                  _full_spec((768, HID)), _full_spec((1, HID)),
                  _full_spec((10, HID)), _full_spec((1, HID)),
                  _full_spec((300, HID)), _full_spec((1, HID)),
                  _full_spec((4 * HID, HID))],
        out_specs=_pk_spec,
        out_shape=jax.ShapeDtypeStruct((NPROWS, PK * HID), jnp.float32),
    )(cx, bx, px, sx, degp, Wc, bc, Wb, bb, Wp, bp, Ws, bs, Wg1)


def _tc2_call(accp, g, degp, b4, W4):
    return pl.pallas_call(
        _tc2_body,
        grid=(GRID,),
        in_specs=[_pk2_spec, _pk_spec, _pk2_spec,
                  _full_spec((1, PK * HID)),
                  _full_spec((PK * HID, PK * HID))],
        out_specs=_pk_spec,
        out_shape=jax.ShapeDtypeStruct((NPROWS, PK * HID), jnp.float32),
    )(accp, g, degp, b4, W4)


def _tc3_call(accp, g, degp, batchq, b4, Wl1, bl1, Wl2, bl2):
    return pl.pallas_call(
        _tc3_body,
        grid=(GRID,),
        in_specs=[_pk2_spec, _pk_spec, _pk2_spec,
                  pl.BlockSpec((1, PK, 1, BP), lambda i: (i, 0, 0, 0)),
                  _full_spec((1, PK * HID)), _full_spec((HID, HID)),
                  _full_spec((1, HID)), _full_spec((HID, OUT)),
                  _full_spec((1, OUT))],
        out_specs=_full_spec((NUM_GRAPHS, OUT)),
        out_shape=jax.ShapeDtypeStruct((NUM_GRAPHS, OUT), jnp.float32),
        scratch_shapes=[pltpu.VMEM((NUM_GRAPHS, HID), jnp.float32),
                        pltpu.VMEM((NUM_GRAPHS, NUM_GRAPHS), jnp.float32)],
    )(accp, g, degp, batchq, b4, Wl1, bl1, Wl2, bl2)


def kernel(content_x, bert_x, profile_x, spacy_x, edge_index, batch,
           Wc, bc, Wb, bb, Wp, bp, Ws, bs,
           Wg1, bg1, Wg2, bg2, Wl1, bl1, Wl2, bl2):
    ei = (1024 * (edge_index // 1024) + 4 * (edge_index % 256)
          + (edge_index // 256) % 4)
    padcols = jnp.broadcast_to(jnp.array([[0], [NBPAD]], jnp.int32),
                               (2, EPAD - E))
    srcdst = jnp.concatenate([ei, padcols], axis=1
                             ).reshape(2, NTILES, NCH, CH)
    zeros32 = jnp.zeros((ZROWS, HID), jnp.float32)
    ones32 = jnp.ones((CH, HID), jnp.float32)
    b1 = bc.reshape(1, HID)
    b2 = bb.reshape(1, HID)
    b3 = bp.reshape(1, HID)
    b4 = bs.reshape(1, HID)
    bg1p = jnp.tile(bg1, PK).reshape(1, PK * HID)
    bg2p = jnp.tile(bg2, PK).reshape(1, PK * HID)
    W4g2 = jnp.kron(jnp.eye(PK, dtype=jnp.float32), Wg2)
    batchp = jnp.concatenate(
        [batch, jnp.full((NBPAD - N,), NUM_GRAPHS, jnp.int32)])
    batchq = batchp.reshape(GRID, PK, 1, BP)

    degp = _deg_kernel(srcdst, zeros32, ones32
                       ).reshape(2, NPROWS, PK * HID)
    g1p = _tc1_call(content_x, bert_x, profile_x, spacy_x, degp,
                    Wc, b1, Wb, b2, Wp, b3, Ws, b4, Wg1)
    acc1 = _edge_kernel(g1p.reshape(ACC_ROWS, HID), srcdst, zeros32
                        ).reshape(2, NPROWS, PK * HID)
    g2p = _tc2_call(acc1, g1p, degp, bg1p, W4g2)
    acc2 = _edge_kernel(g2p.reshape(ACC_ROWS, HID), srcdst, zeros32
                        ).reshape(2, NPROWS, PK * HID)
    return _tc3_call(acc2, g2p, degp, batchq, bg2p,
                     Wl1, bl1.reshape(1, HID), Wl2, bl2.reshape(1, OUT))


# speedup vs baseline: 1.3987x; 1.0892x over previous
import functools

import jax
import jax.numpy as jnp
from jax import lax
from jax.experimental import pallas as pl
from jax.experimental.pallas import tpu as pltpu
from jax.experimental.pallas import tpu_sc as plsc

N = 50000
E = 800000
HID = 32
OUT = 4
NUM_GRAPHS = 128

NTILES = 32
CH = 128
NCH = 196
EPT = NCH * CH
EPAD = NTILES * EPT
KC = 28
NST = NCH // KC
PK = 4
BN = 1024
BP = BN // PK
GRID = -(-N // BN)
NBPAD = GRID * BN

ACC_ROWS = 50304
ZROWS = ACC_ROWS // 16
NPROWS = ACC_ROWS // PK

_mesh = plsc.VectorSubcoreMesh(core_axis_name="c", subcore_axis_name="s")
_sc_params = pltpu.CompilerParams(use_tc_tiling_on_sc=False)


@functools.partial(
    pl.kernel, mesh=_mesh,
    out_type=jax.ShapeDtypeStruct((2, ACC_ROWS, HID), jnp.float32),
    scratch_types=[
        pltpu.VMEM((KC, CH), jnp.int32),
        pltpu.VMEM((CH, HID), jnp.float32),
        pltpu.VMEM_SHARED((ACC_ROWS, HID), jnp.float32),
        pltpu.SemaphoreType.DMA,
    ],
    compiler_params=_sc_params,
)
def _deg_kernel(srcdst_hbm, zeros_hbm, ones_hbm, out_hbm, dstv, onesv, deg,
                sems):
    c = lax.axis_index("c")
    s = lax.axis_index("s")
    wid = c * 16 + s
    pltpu.sync_copy(ones_hbm, onesv)
    pltpu.sync_copy(zeros_hbm, deg.at[pl.ds(s * ZROWS, ZROWS)])
    plsc.subcore_barrier()

    def outer(t, carry):
        pltpu.sync_copy(srcdst_hbm.at[1, wid, pl.ds(t * KC, KC)], dstv)

        def fire(j, cc):
            pltpu.async_copy(onesv, deg.at[dstv.at[j]], sems, add=True)
            return cc

        lax.fori_loop(0, KC, fire, carry)

        def drain(j, cc):
            pltpu.make_async_copy(onesv, deg.at[dstv.at[j]], sems).wait()
            return cc

        return lax.fori_loop(0, KC, drain, carry)

    lax.fori_loop(0, NST, outer, 0)
    plsc.subcore_barrier()
    pltpu.sync_copy(deg.at[pl.ds(s * ZROWS, ZROWS)],
                    out_hbm.at[c, pl.ds(s * ZROWS, ZROWS)])


@functools.partial(
    pl.kernel, mesh=_mesh,
    out_type=jax.ShapeDtypeStruct((2, ACC_ROWS, HID), jnp.float32),
    scratch_types=[
        pltpu.VMEM((KC, CH), jnp.int32),
        pltpu.VMEM((KC, CH), jnp.int32),
        pltpu.VMEM((CH, HID), jnp.float32),
        pltpu.VMEM((CH, HID), jnp.float32),
        pltpu.VMEM((CH, HID), jnp.float32),
        pltpu.VMEM((CH, HID), jnp.float32),
        pltpu.VMEM_SHARED((ACC_ROWS, HID), jnp.float32),
        pltpu.SemaphoreType.DMA,
        pltpu.SemaphoreType.DMA,
        pltpu.SemaphoreType.DMA,
        pltpu.SemaphoreType.DMA,
        pltpu.SemaphoreType.DMA,
        pltpu.SemaphoreType.DMA,
        pltpu.SemaphoreType.DMA,
        pltpu.SemaphoreType.DMA,
    ],
    compiler_params=_sc_params,
)
def _edge_kernel(g_hbm, srcdst_hbm, zeros_hbm, out_hbm,
                 srcv, dstv, r0, r1, r2, r3,
                 acc, sg0, sg1, sg2, sg3, ss0, ss1, ss2, ss3):
    c = lax.axis_index("c")
    s = lax.axis_index("s")
    wid = c * 16 + s
    pltpu.sync_copy(zeros_hbm, acc.at[pl.ds(s * ZROWS, ZROWS)])
    plsc.subcore_barrier()
    rows = [r0, r1, r2, r3]
    sg = [sg0, sg1, sg2, sg3]
    ss = [ss0, ss1, ss2, ss3]

    def outer(t, carry):
        pltpu.sync_copy(srcdst_hbm.at[0, wid, pl.ds(t * KC, KC)], srcv)
        pltpu.sync_copy(srcdst_hbm.at[1, wid, pl.ds(t * KC, KC)], dstv)
        pltpu.async_copy(g_hbm.at[srcv.at[0]], rows[0], sg[0])
        pltpu.async_copy(g_hbm.at[srcv.at[1]], rows[1], sg[1])

        def body(u, cc):
            for b in range(4):
                j = 4 * u + b
                bn = (b + 2) % 4

                @pl.when(j >= 2)
                def _(j=j, bn=bn):
                    pltpu.make_async_copy(rows[bn], acc.at[dstv.at[j - 2]],
                                          ss[bn]).wait()

                @pl.when(j + 2 < KC)
                def _(j=j, bn=bn):
                    pltpu.async_copy(g_hbm.at[srcv.at[j + 2]], rows[bn],
                                     sg[bn])

                pltpu.make_async_copy(g_hbm.at[srcv.at[j]], rows[b],
                                      sg[b]).wait()
                pltpu.async_copy(rows[b], acc.at[dstv.at[j]], ss[b], add=True)
            return cc

        lax.fori_loop(0, KC // 4, body, carry)
        pltpu.make_async_copy(rows[2], acc.at[dstv.at[KC - 2]], ss[2]).wait()
        pltpu.make_async_copy(rows[3], acc.at[dstv.at[KC - 1]], ss[3]).wait()
        return carry

    lax.fori_loop(0, NST, outer, 0)
    plsc.subcore_barrier()
    pltpu.sync_copy(acc.at[pl.ds(s * ZROWS, ZROWS)],
                    out_hbm.at[c, pl.ds(s * ZROWS, ZROWS)])


def _dinvp_from(degp):
    return lax.rsqrt(degp[0] + degp[1] + 1.0)


def _tc1_body(cx, bx, px, sx, degp, Wc, bc, Wb, bb, Wp, bp, Ws, bs, Wg1,
              g1_out):
    ch = jnp.maximum(jnp.dot(cx[...], Wc[...],
                             preferred_element_type=jnp.float32) + bc[...], 0.0)
    bh = jnp.maximum(jnp.dot(bx[...], Wb[...],
                             preferred_element_type=jnp.float32) + bb[...], 0.0)
    ph = jnp.maximum(jnp.dot(px[...], Wp[...],
                             preferred_element_type=jnp.float32) + bp[...], 0.0)
    sh = jnp.maximum(jnp.dot(sx[...], Ws[...],
                             preferred_element_type=jnp.float32) + bs[...], 0.0)
    x0 = jnp.concatenate([ch, bh, ph, sh], axis=1)
    h1 = jnp.dot(x0, Wg1[...], preferred_element_type=jnp.float32)
    h1p = jnp.concatenate([h1[k * BP:(k + 1) * BP] for k in range(PK)],
                          axis=1)
    g1_out[...] = h1p * _dinvp_from(degp[...])


def _tc2_body(accp, g, degp, b4, W4, g_out):
    dinvp = _dinvp_from(degp[...])
    acc = accp[0] + accp[1]
    x = jnp.maximum(dinvp * (acc + g[...]) + b4[...], 0.0)
    g_out[...] = jnp.dot(x, W4[...], preferred_element_type=jnp.float32) * dinvp


def _tc3_body(accp, g, degp, batchq, b4, Wl1, bl1, Wl2, bl2, out,
              sums, counts):
    i = pl.program_id(0)

    @pl.when(i == 0)
    def _():
        sums[...] = jnp.zeros_like(sums)
        counts[...] = jnp.zeros_like(counts)

    dinvp = _dinvp_from(degp[...])
    acc = accp[0] + accp[1]
    x2 = jnp.maximum(dinvp * (acc + g[...]) + b4[...], 0.0)
    node = (i * BN + lax.broadcasted_iota(jnp.int32, (BP, PK * HID), 0)
            + BP * (lax.broadcasted_iota(jnp.int32, (BP, PK * HID), 1) // HID))
    x2 = jnp.where(node < N, x2, 0.0)
    giota = lax.broadcasted_iota(jnp.int32, (NUM_GRAPHS, BP), 0)
    for q in range(PK):
        onehot = (giota == batchq[0, q]).astype(jnp.float32)
        sums[...] += jnp.dot(onehot, x2[:, q * HID:(q + 1) * HID],
                             preferred_element_type=jnp.float32)
        counts[...] += jnp.sum(onehot, axis=1, keepdims=True)

    @pl.when(i == GRID - 1)
    def _():
        pooled = sums[...] / jnp.maximum(counts[:, 0:1], 1.0)
        h = jnp.maximum(jnp.dot(pooled, Wl1[...],
                                preferred_element_type=jnp.float32) + bl1[...],
                        0.0)
        logits = jnp.dot(h, Wl2[...],
                         preferred_element_type=jnp.float32) + bl2[...]
        m = jnp.max(logits, axis=-1, keepdims=True)
        sh = logits - m
        out[...] = sh - jnp.log(jnp.sum(jnp.exp(sh), axis=-1, keepdims=True))


def _row_spec(w):
    return pl.BlockSpec((BN, w), lambda i: (i, 0))


def _full_spec(shape):
    return pl.BlockSpec(shape, lambda i: tuple(0 for _ in shape))


_pk_spec = pl.BlockSpec((BP, PK * HID), lambda i: (i, 0))
_pk2_spec = pl.BlockSpec((2, BP, PK * HID), lambda i: (0, i, 0))


def _tc1_call(cx, bx, px, sx, degp, Wc, bc, Wb, bb, Wp, bp, Ws, bs, Wg1):
    return pl.pallas_call(
        _tc1_body,
        grid=(GRID,),
        in_specs=[_row_spec(310), _row_spec(768), _row_spec(10), _row_spec(300),
                  _pk2_spec,
                  _full_spec((310, HID)), _full_spec((1, HID)),
                  _full_spec((768, HID)), _full_spec((1, HID)),
                  _full_spec((10, HID)), _full_spec((1, HID)),
                  _full_spec((300, HID)), _full_spec((1, HID)),
                  _full_spec((4 * HID, HID))],
        out_specs=_pk_spec,
        out_shape=jax.ShapeDtypeStruct((NPROWS, PK * HID), jnp.float32),
    )(cx, bx, px, sx, degp, Wc, bc, Wb, bb, Wp, bp, Ws, bs, Wg1)


def _tc2_call(accp, g, degp, b4, W4):
    return pl.pallas_call(
        _tc2_body,
        grid=(GRID,),
        in_specs=[_pk2_spec, _pk_spec, _pk2_spec,
                  _full_spec((1, PK * HID)),
                  _full_spec((PK * HID, PK * HID))],
        out_specs=_pk_spec,
        out_shape=jax.ShapeDtypeStruct((NPROWS, PK * HID), jnp.float32),
    )(accp, g, degp, b4, W4)


def _tc3_call(accp, g, degp, batchq, b4, Wl1, bl1, Wl2, bl2):
    return pl.pallas_call(
        _tc3_body,
        grid=(GRID,),
        in_specs=[_pk2_spec, _pk_spec, _pk2_spec,
                  pl.BlockSpec((1, PK, 1, BP), lambda i: (i, 0, 0, 0)),
                  _full_spec((1, PK * HID)), _full_spec((HID, HID)),
                  _full_spec((1, HID)), _full_spec((HID, OUT)),
                  _full_spec((1, OUT))],
        out_specs=_full_spec((NUM_GRAPHS, OUT)),
        out_shape=jax.ShapeDtypeStruct((NUM_GRAPHS, OUT), jnp.float32),
        scratch_shapes=[pltpu.VMEM((NUM_GRAPHS, HID), jnp.float32),
                        pltpu.VMEM((NUM_GRAPHS, NUM_GRAPHS), jnp.float32)],
    )(accp, g, degp, batchq, b4, Wl1, bl1, Wl2, bl2)


def kernel(content_x, bert_x, profile_x, spacy_x, edge_index, batch,
           Wc, bc, Wb, bb, Wp, bp, Ws, bs,
           Wg1, bg1, Wg2, bg2, Wl1, bl1, Wl2, bl2):
    ei = (1024 * (edge_index // 1024) + 4 * (edge_index % 256)
          + (edge_index // 256) % 4)
    padcols = jnp.broadcast_to(jnp.array([[0], [NBPAD]], jnp.int32),
                               (2, EPAD - E))
    srcdst = jnp.concatenate([ei, padcols], axis=1
                             ).reshape(2, NTILES, NCH, CH)
    zeros32 = jnp.zeros((ZROWS, HID), jnp.float32)
    ones32 = jnp.ones((CH, HID), jnp.float32)
    b1 = bc.reshape(1, HID)
    b2 = bb.reshape(1, HID)
    b3 = bp.reshape(1, HID)
    b4 = bs.reshape(1, HID)
    bg1p = jnp.tile(bg1, PK).reshape(1, PK * HID)
    bg2p = jnp.tile(bg2, PK).reshape(1, PK * HID)
    W4g2 = jnp.kron(jnp.eye(PK, dtype=jnp.float32), Wg2)
    batchp = jnp.concatenate(
        [batch, jnp.full((NBPAD - N,), NUM_GRAPHS, jnp.int32)])
    batchq = batchp.reshape(GRID, PK, 1, BP)

    degp = _deg_kernel(srcdst, zeros32, ones32
                       ).reshape(2, NPROWS, PK * HID)
    g1p = _tc1_call(content_x, bert_x, profile_x, spacy_x, degp,
                    Wc, b1, Wb, b2, Wp, b3, Ws, b4, Wg1)
    acc1 = _edge_kernel(g1p.reshape(ACC_ROWS, HID), srcdst, zeros32
                        ).reshape(2, NPROWS, PK * HID)
    g2p = _tc2_call(acc1, g1p, degp, bg1p, W4g2)
    acc2 = _edge_kernel(g2p.reshape(ACC_ROWS, HID), srcdst, zeros32
                        ).reshape(2, NPROWS, PK * HID)
    return _tc3_call(acc2, g2p, degp, batchq, bg2p,
                     Wl1, bl1.reshape(1, HID), Wl2, bl2.reshape(1, OUT))


# speedup vs baseline: 1.6599x; 1.1868x over previous
import functools

import jax
import jax.numpy as jnp
from jax import lax
from jax.experimental import pallas as pl
from jax.experimental.pallas import tpu as pltpu
from jax.experimental.pallas import tpu_sc as plsc

N = 50000
E = 800000
HID = 32
OUT = 4
NUM_GRAPHS = 128

NTILES = 32
CH = 128
NCH = 196
EPT = NCH * CH
EPAD = NTILES * EPT
KC = 28
NST = NCH // KC
PK = 4
BN = 1024
BP = BN // PK
GRID = -(-N // BN)
NBPAD = GRID * BN

ACC_ROWS = 50304
ZROWS = ACC_ROWS // 16
NPROWS = ACC_ROWS // PK

_mesh = plsc.VectorSubcoreMesh(core_axis_name="c", subcore_axis_name="s")
_sc_params = pltpu.CompilerParams(use_tc_tiling_on_sc=False)


@functools.partial(
    pl.kernel, mesh=_mesh,
    out_type=jax.ShapeDtypeStruct((2, ACC_ROWS, HID), jnp.float32),
    scratch_types=[
        pltpu.VMEM((KC, CH), jnp.int32),
        pltpu.VMEM((CH, HID), jnp.float32),
        pltpu.VMEM_SHARED((ACC_ROWS, HID), jnp.float32),
        pltpu.SemaphoreType.DMA,
    ],
    compiler_params=_sc_params,
)
def _deg_kernel(srcdst_hbm, zeros_hbm, ones_hbm, out_hbm, dstv, onesv, deg,
                sems):
    c = lax.axis_index("c")
    s = lax.axis_index("s")
    wid = c * 16 + s
    pltpu.sync_copy(ones_hbm, onesv)
    pltpu.sync_copy(zeros_hbm, deg.at[pl.ds(s * ZROWS, ZROWS)])
    plsc.subcore_barrier()

    def outer(t, carry):
        pltpu.sync_copy(srcdst_hbm.at[1, wid, pl.ds(t * KC, KC)], dstv)

        def fire(j, cc):
            pltpu.async_copy(onesv, deg.at[dstv.at[j]], sems, add=True)
            return cc

        lax.fori_loop(0, KC, fire, carry)

        def drain(j, cc):
            pltpu.make_async_copy(onesv, deg.at[dstv.at[j]], sems).wait()
            return cc

        return lax.fori_loop(0, KC, drain, carry)

    lax.fori_loop(0, NST, outer, 0)
    plsc.subcore_barrier()
    pltpu.sync_copy(deg.at[pl.ds(s * ZROWS, ZROWS)],
                    out_hbm.at[c, pl.ds(s * ZROWS, ZROWS)])


@functools.partial(
    pl.kernel, mesh=_mesh,
    out_type=jax.ShapeDtypeStruct((2, ACC_ROWS, HID), jnp.float32),
    scratch_types=[
        pltpu.VMEM((KC, CH), jnp.int32),
        pltpu.VMEM((KC, CH), jnp.int32),
        pltpu.VMEM((CH, HID), jnp.float32),
        pltpu.VMEM((CH, HID), jnp.float32),
        pltpu.VMEM((CH, HID), jnp.float32),
        pltpu.VMEM((CH, HID), jnp.float32),
        pltpu.VMEM_SHARED((ACC_ROWS, HID), jnp.float32),
        pltpu.SemaphoreType.DMA,
        pltpu.SemaphoreType.DMA,
        pltpu.SemaphoreType.DMA,
        pltpu.SemaphoreType.DMA,
        pltpu.SemaphoreType.DMA,
        pltpu.SemaphoreType.DMA,
        pltpu.SemaphoreType.DMA,
        pltpu.SemaphoreType.DMA,
    ],
    compiler_params=_sc_params,
)
def _edge_kernel(g_hbm, srcdst_hbm, zeros_hbm, out_hbm,
                 srcv, dstv, r0, r1, r2, r3,
                 acc, sg0, sg1, sg2, sg3, ss0, ss1, ss2, ss3):
    c = lax.axis_index("c")
    s = lax.axis_index("s")
    wid = c * 16 + s
    pltpu.sync_copy(zeros_hbm, acc.at[pl.ds(s * ZROWS, ZROWS)])
    plsc.subcore_barrier()
    rows = [r0, r1, r2, r3]
    sg = [sg0, sg1, sg2, sg3]
    ss = [ss0, ss1, ss2, ss3]

    def outer(t, carry):
        pltpu.sync_copy(srcdst_hbm.at[0, wid, pl.ds(t * KC, KC)], srcv)
        pltpu.sync_copy(srcdst_hbm.at[1, wid, pl.ds(t * KC, KC)], dstv)
        pltpu.async_copy(g_hbm.at[srcv.at[0]], rows[0], sg[0])
        pltpu.async_copy(g_hbm.at[srcv.at[1]], rows[1], sg[1])

        def body(u, cc):
            for b in range(4):
                j = 4 * u + b
                bn = (b + 2) % 4

                @pl.when(j >= 2)
                def _(j=j, bn=bn):
                    pltpu.make_async_copy(rows[bn], acc.at[dstv.at[j - 2]],
                                          ss[bn]).wait()

                @pl.when(j + 2 < KC)
                def _(j=j, bn=bn):
                    pltpu.async_copy(g_hbm.at[srcv.at[j + 2]], rows[bn],
                                     sg[bn])

                pltpu.make_async_copy(g_hbm.at[srcv.at[j]], rows[b],
                                      sg[b]).wait()
                pltpu.async_copy(rows[b], acc.at[dstv.at[j]], ss[b], add=True)
            return cc

        lax.fori_loop(0, KC // 4, body, carry)
        pltpu.make_async_copy(rows[2], acc.at[dstv.at[KC - 2]], ss[2]).wait()
        pltpu.make_async_copy(rows[3], acc.at[dstv.at[KC - 1]], ss[3]).wait()
        return carry

    lax.fori_loop(0, NST, outer, 0)
    plsc.subcore_barrier()
    pltpu.sync_copy(acc.at[pl.ds(s * ZROWS, ZROWS)],
                    out_hbm.at[c, pl.ds(s * ZROWS, ZROWS)])


def _dinvp_from(degp):
    return lax.rsqrt(degp[0] + degp[1] + 1.0)


def _dot_t(xt, w):
    return lax.dot_general(xt, w, (((0,), (0,)), ((), ())),
                           preferred_element_type=jnp.float32)


def _tc1_body(cx, bx, px, sx, degp, Wc, bc, Wb, bb, Wp, bp, Ws, bs, Wg1,
              g1_out):
    ch = jnp.maximum(_dot_t(cx[...], Wc[...]) + bc[...], 0.0)
    bh = jnp.maximum(jnp.dot(bx[...], Wb[...],
                             preferred_element_type=jnp.float32) + bb[...], 0.0)
    ph = jnp.maximum(_dot_t(px[...], Wp[...]) + bp[...], 0.0)
    sh = jnp.maximum(_dot_t(sx[...], Ws[...]) + bs[...], 0.0)
    x0 = jnp.concatenate([ch, bh, ph, sh], axis=1)
    h1 = jnp.dot(x0, Wg1[...], preferred_element_type=jnp.float32)
    h1p = jnp.concatenate([h1[k * BP:(k + 1) * BP] for k in range(PK)],
                          axis=1)
    g1_out[...] = h1p * _dinvp_from(degp[...])


def _tc2_body(accp, g, degp, b4, W4, g_out):
    dinvp = _dinvp_from(degp[...])
    acc = accp[0] + accp[1]
    x = jnp.maximum(dinvp * (acc + g[...]) + b4[...], 0.0)
    g_out[...] = jnp.dot(x, W4[...], preferred_element_type=jnp.float32) * dinvp


def _tc3_body(accp, g, degp, batchq, b4, Wl1, bl1, Wl2, bl2, out,
              sums, counts):
    i = pl.program_id(0)

    @pl.when(i == 0)
    def _():
        sums[...] = jnp.zeros_like(sums)
        counts[...] = jnp.zeros_like(counts)

    dinvp = _dinvp_from(degp[...])
    acc = accp[0] + accp[1]
    x2 = jnp.maximum(dinvp * (acc + g[...]) + b4[...], 0.0)
    node = (i * BN + lax.broadcasted_iota(jnp.int32, (BP, PK * HID), 0)
            + BP * (lax.broadcasted_iota(jnp.int32, (BP, PK * HID), 1) // HID))
    x2 = jnp.where(node < N, x2, 0.0)
    giota = lax.broadcasted_iota(jnp.int32, (NUM_GRAPHS, BP), 0)
    for q in range(PK):
        onehot = (giota == batchq[0, q]).astype(jnp.float32)
        sums[...] += jnp.dot(onehot, x2[:, q * HID:(q + 1) * HID],
                             preferred_element_type=jnp.float32)
        counts[...] += jnp.sum(onehot, axis=1, keepdims=True)

    @pl.when(i == GRID - 1)
    def _():
        pooled = sums[...] / jnp.maximum(counts[:, 0:1], 1.0)
        h = jnp.maximum(jnp.dot(pooled, Wl1[...],
                                preferred_element_type=jnp.float32) + bl1[...],
                        0.0)
        logits = jnp.dot(h, Wl2[...],
                         preferred_element_type=jnp.float32) + bl2[...]
        m = jnp.max(logits, axis=-1, keepdims=True)
        sh = logits - m
        out[...] = sh - jnp.log(jnp.sum(jnp.exp(sh), axis=-1, keepdims=True))


def _row_spec(w):
    return pl.BlockSpec((BN, w), lambda i: (i, 0))


def _full_spec(shape):
    return pl.BlockSpec(shape, lambda i: tuple(0 for _ in shape))


_pk_spec = pl.BlockSpec((BP, PK * HID), lambda i: (i, 0))
_pk2_spec = pl.BlockSpec((2, BP, PK * HID), lambda i: (0, i, 0))


def _tc1_call(cx, bx, px, sx, degp, Wc, bc, Wb, bb, Wp, bp, Ws, bs, Wg1):
    return pl.pallas_call(
        _tc1_body,
        grid=(GRID,),
        in_specs=[pl.BlockSpec((310, BN), lambda i: (0, i)),
                  _row_spec(768),
                  pl.BlockSpec((10, BN), lambda i: (0, i)),
                  pl.BlockSpec((300, BN), lambda i: (0, i)),
                  _pk2_spec,
                  _full_spec((310, HID)), _full_spec((1, HID)),
                  _full_spec((768, HID)), _full_spec((1, HID)),
                  _full_spec((10, HID)), _full_spec((1, HID)),
                  _full_spec((300, HID)), _full_spec((1, HID)),
                  _full_spec((4 * HID, HID))],
        out_specs=_pk_spec,
        out_shape=jax.ShapeDtypeStruct((NPROWS, PK * HID), jnp.float32),
    )(cx, bx, px, sx, degp, Wc, bc, Wb, bb, Wp, bp, Ws, bs, Wg1)


def _tc2_call(accp, g, degp, b4, W4):
    bp2 = NPROWS // 4
    return pl.pallas_call(
        _tc2_body,
        grid=(4,),
        in_specs=[pl.BlockSpec((2, bp2, PK * HID), lambda i: (0, i, 0)),
                  pl.BlockSpec((bp2, PK * HID), lambda i: (i, 0)),
                  pl.BlockSpec((2, bp2, PK * HID), lambda i: (0, i, 0)),
                  _full_spec((1, PK * HID)),
                  _full_spec((PK * HID, PK * HID))],
        out_specs=pl.BlockSpec((bp2, PK * HID), lambda i: (i, 0)),
        out_shape=jax.ShapeDtypeStruct((NPROWS, PK * HID), jnp.float32),
    )(accp, g, degp, b4, W4)


def _tc3_call(accp, g, degp, batchq, b4, Wl1, bl1, Wl2, bl2):
    return pl.pallas_call(
        _tc3_body,
        grid=(GRID,),
        in_specs=[_pk2_spec, _pk_spec, _pk2_spec,
                  pl.BlockSpec((1, PK, 1, BP), lambda i: (i, 0, 0, 0)),
                  _full_spec((1, PK * HID)), _full_spec((HID, HID)),
                  _full_spec((1, HID)), _full_spec((HID, OUT)),
                  _full_spec((1, OUT))],
        out_specs=_full_spec((NUM_GRAPHS, OUT)),
        out_shape=jax.ShapeDtypeStruct((NUM_GRAPHS, OUT), jnp.float32),
        scratch_shapes=[pltpu.VMEM((NUM_GRAPHS, HID), jnp.float32),
                        pltpu.VMEM((NUM_GRAPHS, NUM_GRAPHS), jnp.float32)],
    )(accp, g, degp, batchq, b4, Wl1, bl1, Wl2, bl2)


def kernel(content_x, bert_x, profile_x, spacy_x, edge_index, batch,
           Wc, bc, Wb, bb, Wp, bp, Ws, bs,
           Wg1, bg1, Wg2, bg2, Wl1, bl1, Wl2, bl2):
    ei = (1024 * (edge_index // 1024) + 4 * (edge_index % 256)
          + (edge_index // 256) % 4)
    padcols = jnp.broadcast_to(jnp.array([[0], [NBPAD]], jnp.int32),
                               (2, EPAD - E))
    srcdst = jnp.concatenate([ei, padcols], axis=1
                             ).reshape(2, NTILES, NCH, CH)
    zeros32 = jnp.zeros((ZROWS, HID), jnp.float32)
    ones32 = jnp.ones((CH, HID), jnp.float32)
    b1 = bc.reshape(1, HID)
    b2 = bb.reshape(1, HID)
    b3 = bp.reshape(1, HID)
    b4 = bs.reshape(1, HID)
    bg1p = jnp.tile(bg1, PK).reshape(1, PK * HID)
    bg2p = jnp.tile(bg2, PK).reshape(1, PK * HID)
    W4g2 = jnp.kron(jnp.eye(PK, dtype=jnp.float32), Wg2)
    batchp = jnp.concatenate(
        [batch, jnp.full((NBPAD - N,), NUM_GRAPHS, jnp.int32)])
    batchq = batchp.reshape(GRID, PK, 1, BP)

    degp = _deg_kernel(srcdst, zeros32, ones32
                       ).reshape(2, NPROWS, PK * HID)
    g1p = _tc1_call(content_x.T, bert_x, profile_x.T, spacy_x.T, degp,
                    Wc, b1, Wb, b2, Wp, b3, Ws, b4, Wg1)
    acc1 = _edge_kernel(g1p.reshape(ACC_ROWS, HID), srcdst, zeros32
                        ).reshape(2, NPROWS, PK * HID)
    g2p = _tc2_call(acc1, g1p, degp, bg1p, W4g2)
    acc2 = _edge_kernel(g2p.reshape(ACC_ROWS, HID), srcdst, zeros32
                        ).reshape(2, NPROWS, PK * HID)
    return _tc3_call(acc2, g2p, degp, batchq, bg2p,
                     Wl1, bl1.reshape(1, HID), Wl2, bl2.reshape(1, OUT))


# speedup vs baseline: 1.7950x; 1.0814x over previous
import functools

import jax
import jax.numpy as jnp
from jax import lax
from jax.experimental import pallas as pl
from jax.experimental.pallas import tpu as pltpu
from jax.experimental.pallas import tpu_sc as plsc

N = 50000
E = 800000
HID = 32
OUT = 4
NUM_GRAPHS = 128

NTILES = 32
CH = 128
NCH = 196
EPT = NCH * CH
EPAD = NTILES * EPT
KC = 28
NST = NCH // KC
PK = 4
BN = 1024
BP = BN // PK
GRID = -(-N // BN)
NBPAD = GRID * BN

ACC_ROWS = 50304
ZROWS = ACC_ROWS // 16
NPROWS = ACC_ROWS // PK

_mesh = plsc.VectorSubcoreMesh(core_axis_name="c", subcore_axis_name="s")
_sc_params = pltpu.CompilerParams(use_tc_tiling_on_sc=False)


@functools.partial(
    pl.kernel, mesh=_mesh,
    out_type=jax.ShapeDtypeStruct((2, ACC_ROWS, HID), jnp.float32),
    scratch_types=[
        pltpu.VMEM((KC, CH), jnp.int32),
        pltpu.VMEM((CH, HID), jnp.float32),
        pltpu.VMEM_SHARED((ACC_ROWS, HID), jnp.float32),
        pltpu.SemaphoreType.DMA,
    ],
    compiler_params=_sc_params,
)
def _deg_kernel(srcdst_hbm, zeros_hbm, ones_hbm, out_hbm, dstv, onesv, deg,
                sems):
    c = lax.axis_index("c")
    s = lax.axis_index("s")
    wid = c * 16 + s
    pltpu.sync_copy(ones_hbm, onesv)
    pltpu.sync_copy(zeros_hbm, deg.at[pl.ds(s * ZROWS, ZROWS)])
    plsc.subcore_barrier()

    def outer(t, carry):
        pltpu.sync_copy(srcdst_hbm.at[1, wid, pl.ds(t * KC, KC)], dstv)

        def fire(j, cc):
            pltpu.async_copy(onesv, deg.at[dstv.at[j]], sems, add=True)
            return cc

        lax.fori_loop(0, KC, fire, carry)

        def drain(j, cc):
            pltpu.make_async_copy(onesv, deg.at[dstv.at[j]], sems).wait()
            return cc

        return lax.fori_loop(0, KC, drain, carry)

    lax.fori_loop(0, NST, outer, 0)
    plsc.subcore_barrier()
    pltpu.sync_copy(deg.at[pl.ds(s * ZROWS, ZROWS)],
                    out_hbm.at[c, pl.ds(s * ZROWS, ZROWS)])


@functools.partial(
    pl.kernel, mesh=_mesh,
    out_type=jax.ShapeDtypeStruct((2, ACC_ROWS, HID), jnp.float32),
    scratch_types=[
        pltpu.VMEM((KC, CH), jnp.int32),
        pltpu.VMEM((KC, CH), jnp.int32),
        pltpu.VMEM((CH, HID), jnp.float32),
        pltpu.VMEM((CH, HID), jnp.float32),
        pltpu.VMEM((CH, HID), jnp.float32),
        pltpu.VMEM((CH, HID), jnp.float32),
        pltpu.VMEM_SHARED((ACC_ROWS, HID), jnp.float32),
        pltpu.SemaphoreType.DMA,
        pltpu.SemaphoreType.DMA,
        pltpu.SemaphoreType.DMA,
        pltpu.SemaphoreType.DMA,
        pltpu.SemaphoreType.DMA,
        pltpu.SemaphoreType.DMA,
        pltpu.SemaphoreType.DMA,
        pltpu.SemaphoreType.DMA,
    ],
    compiler_params=_sc_params,
)
def _edge_kernel(g_hbm, srcdst_hbm, zeros_hbm, out_hbm,
                 srcv, dstv, r0, r1, r2, r3,
                 acc, sg0, sg1, sg2, sg3, ss0, ss1, ss2, ss3):
    c = lax.axis_index("c")
    s = lax.axis_index("s")
    wid = c * 16 + s
    pltpu.sync_copy(zeros_hbm, acc.at[pl.ds(s * ZROWS, ZROWS)])
    plsc.subcore_barrier()
    rows = [r0, r1, r2, r3]
    sg = [sg0, sg1, sg2, sg3]
    ss = [ss0, ss1, ss2, ss3]

    def outer(t, carry):
        pltpu.sync_copy(srcdst_hbm.at[0, wid, pl.ds(t * KC, KC)], srcv)
        pltpu.sync_copy(srcdst_hbm.at[1, wid, pl.ds(t * KC, KC)], dstv)
        pltpu.async_copy(g_hbm.at[srcv.at[0]], rows[0], sg[0])
        pltpu.async_copy(g_hbm.at[srcv.at[1]], rows[1], sg[1])

        def body(u, cc):
            for b in range(4):
                j = 4 * u + b
                bn = (b + 2) % 4

                @pl.when(j >= 2)
                def _(j=j, bn=bn):
                    pltpu.make_async_copy(rows[bn], acc.at[dstv.at[j - 2]],
                                          ss[bn]).wait()

                @pl.when(j + 2 < KC)
                def _(j=j, bn=bn):
                    pltpu.async_copy(g_hbm.at[srcv.at[j + 2]], rows[bn],
                                     sg[bn])

                pltpu.make_async_copy(g_hbm.at[srcv.at[j]], rows[b],
                                      sg[b]).wait()
                pltpu.async_copy(rows[b], acc.at[dstv.at[j]], ss[b], add=True)
            return cc

        lax.fori_loop(0, KC // 4, body, carry)
        pltpu.make_async_copy(rows[2], acc.at[dstv.at[KC - 2]], ss[2]).wait()
        pltpu.make_async_copy(rows[3], acc.at[dstv.at[KC - 1]], ss[3]).wait()
        return carry

    lax.fori_loop(0, NST, outer, 0)
    plsc.subcore_barrier()
    pltpu.sync_copy(acc.at[pl.ds(s * ZROWS, ZROWS)],
                    out_hbm.at[c, pl.ds(s * ZROWS, ZROWS)])


def _dinvp_from(degp):
    return lax.rsqrt(degp[0] + degp[1] + 1.0)


def _dot_t(xt, w):
    return lax.dot_general(xt, w, (((0,), (0,)), ((), ())),
                           preferred_element_type=jnp.float32)


def _tc1_body(cx, bx, px, sx, Wc, bc, Wb, bb, Wp, bp, Ws, bs, Wg1,
              h1_out):
    ch = jnp.maximum(_dot_t(cx[...], Wc[...]) + bc[...], 0.0)
    bh = jnp.maximum(jnp.dot(bx[...], Wb[...],
                             preferred_element_type=jnp.float32) + bb[...], 0.0)
    ph = jnp.maximum(_dot_t(px[...], Wp[...]) + bp[...], 0.0)
    sh = jnp.maximum(_dot_t(sx[...], Ws[...]) + bs[...], 0.0)
    x0 = jnp.concatenate([ch, bh, ph, sh], axis=1)
    h1 = jnp.dot(x0, Wg1[...], preferred_element_type=jnp.float32)
    h1_out[...] = jnp.concatenate([h1[k * BP:(k + 1) * BP] for k in range(PK)],
                                  axis=1)


def _scale_body(h, degp, g_out):
    g_out[...] = h[...] * _dinvp_from(degp[...])


def _tc2_body(accp, g, degp, b4, W4, g_out):
    dinvp = _dinvp_from(degp[...])
    acc = accp[0] + accp[1]
    x = jnp.maximum(dinvp * (acc + g[...]) + b4[...], 0.0)
    g_out[...] = jnp.dot(x, W4[...], preferred_element_type=jnp.float32) * dinvp


BP3 = NPROWS // 4


def _tc3_body(accp, g, degp, batchq, b4, Wl1, bl1, Wl2, bl2, out,
              sums, counts):
    i = pl.program_id(0)

    @pl.when(i == 0)
    def _():
        sums[...] = jnp.zeros_like(sums)
        counts[...] = jnp.zeros_like(counts)

    dinvp = _dinvp_from(degp[...])
    acc = accp[0] + accp[1]
    x2 = jnp.maximum(dinvp * (acc + g[...]) + b4[...], 0.0)
    rr = i * BP3 + lax.broadcasted_iota(jnp.int32, (BP3, PK * HID), 0)
    node = (1024 * (rr // 256) + rr % 256
            + 256 * (lax.broadcasted_iota(jnp.int32, (BP3, PK * HID), 1)
                     // HID))
    x2 = jnp.where(node < N, x2, 0.0)
    giota = lax.broadcasted_iota(jnp.int32, (NUM_GRAPHS, BP3), 0)
    for q in range(PK):
        onehot = (giota == batchq[0, q]).astype(jnp.float32)
        sums[...] += jnp.dot(onehot, x2[:, q * HID:(q + 1) * HID],
                             preferred_element_type=jnp.float32)
        counts[...] += jnp.sum(onehot, axis=1, keepdims=True)

    @pl.when(i == 3)
    def _():
        pooled = sums[...] / jnp.maximum(counts[:, 0:1], 1.0)
        h = jnp.maximum(jnp.dot(pooled, Wl1[...],
                                preferred_element_type=jnp.float32) + bl1[...],
                        0.0)
        logits = jnp.dot(h, Wl2[...],
                         preferred_element_type=jnp.float32) + bl2[...]
        m = jnp.max(logits, axis=-1, keepdims=True)
        sh = logits - m
        out[...] = sh - jnp.log(jnp.sum(jnp.exp(sh), axis=-1, keepdims=True))


def _row_spec(w):
    return pl.BlockSpec((BN, w), lambda i: (i, 0))


def _full_spec(shape):
    return pl.BlockSpec(shape, lambda i: tuple(0 for _ in shape))


_pk_spec = pl.BlockSpec((BP, PK * HID), lambda i: (i, 0))
_pk2_spec = pl.BlockSpec((2, BP, PK * HID), lambda i: (0, i, 0))


def _tc1_call(cx, bx, px, sx, Wc, bc, Wb, bb, Wp, bp, Ws, bs, Wg1):
    return pl.pallas_call(
        _tc1_body,
        grid=(GRID,),
        in_specs=[pl.BlockSpec((310, BN), lambda i: (0, i)),
                  _row_spec(768),
                  pl.BlockSpec((10, BN), lambda i: (0, i)),
                  pl.BlockSpec((300, BN), lambda i: (0, i)),
                  _full_spec((310, HID)), _full_spec((1, HID)),
                  _full_spec((768, HID)), _full_spec((1, HID)),
                  _full_spec((10, HID)), _full_spec((1, HID)),
                  _full_spec((300, HID)), _full_spec((1, HID)),
                  _full_spec((4 * HID, HID))],
        out_specs=_pk_spec,
        out_shape=jax.ShapeDtypeStruct((NPROWS, PK * HID), jnp.float32),
    )(cx, bx, px, sx, Wc, bc, Wb, bb, Wp, bp, Ws, bs, Wg1)


def _scale_call(h, degp):
    bp2 = NPROWS // 4
    return pl.pallas_call(
        _scale_body,
        grid=(4,),
        in_specs=[pl.BlockSpec((bp2, PK * HID), lambda i: (i, 0)),
                  pl.BlockSpec((2, bp2, PK * HID), lambda i: (0, i, 0))],
        out_specs=pl.BlockSpec((bp2, PK * HID), lambda i: (i, 0)),
        out_shape=jax.ShapeDtypeStruct((NPROWS, PK * HID), jnp.float32),
    )(h, degp)


def _tc2_call(accp, g, degp, b4, W4):
    bp2 = NPROWS // 4
    return pl.pallas_call(
        _tc2_body,
        grid=(4,),
        in_specs=[pl.BlockSpec((2, bp2, PK * HID), lambda i: (0, i, 0)),
                  pl.BlockSpec((bp2, PK * HID), lambda i: (i, 0)),
                  pl.BlockSpec((2, bp2, PK * HID), lambda i: (0, i, 0)),
                  _full_spec((1, PK * HID)),
                  _full_spec((PK * HID, PK * HID))],
        out_specs=pl.BlockSpec((bp2, PK * HID), lambda i: (i, 0)),
        out_shape=jax.ShapeDtypeStruct((NPROWS, PK * HID), jnp.float32),
    )(accp, g, degp, b4, W4)


def _tc3_call(accp, g, degp, batchq, b4, Wl1, bl1, Wl2, bl2):
    return pl.pallas_call(
        _tc3_body,
        grid=(4,),
        in_specs=[pl.BlockSpec((2, BP3, PK * HID), lambda i: (0, i, 0)),
                  pl.BlockSpec((BP3, PK * HID), lambda i: (i, 0)),
                  pl.BlockSpec((2, BP3, PK * HID), lambda i: (0, i, 0)),
                  pl.BlockSpec((1, PK, 1, BP3), lambda i: (i, 0, 0, 0)),
                  _full_spec((1, PK * HID)), _full_spec((HID, HID)),
                  _full_spec((1, HID)), _full_spec((HID, OUT)),
                  _full_spec((1, OUT))],
        out_specs=_full_spec((NUM_GRAPHS, OUT)),
        out_shape=jax.ShapeDtypeStruct((NUM_GRAPHS, OUT), jnp.float32),
        scratch_shapes=[pltpu.VMEM((NUM_GRAPHS, HID), jnp.float32),
                        pltpu.VMEM((NUM_GRAPHS, NUM_GRAPHS), jnp.float32)],
    )(accp, g, degp, batchq, b4, Wl1, bl1, Wl2, bl2)


def kernel(content_x, bert_x, profile_x, spacy_x, edge_index, batch,
           Wc, bc, Wb, bb, Wp, bp, Ws, bs,
           Wg1, bg1, Wg2, bg2, Wl1, bl1, Wl2, bl2):
    ei = (1024 * (edge_index // 1024) + 4 * (edge_index % 256)
          + (edge_index // 256) % 4)
    padcols = jnp.broadcast_to(jnp.array([[0], [NBPAD]], jnp.int32),
                               (2, EPAD - E))
    srcdst = jnp.concatenate([ei, padcols], axis=1
                             ).reshape(2, NTILES, NCH, CH)
    zeros32 = jnp.zeros((ZROWS, HID), jnp.float32)
    ones32 = jnp.ones((CH, HID), jnp.float32)
    b1 = bc.reshape(1, HID)
    b2 = bb.reshape(1, HID)
    b3 = bp.reshape(1, HID)
    b4 = bs.reshape(1, HID)
    bg1p = jnp.tile(bg1, PK).reshape(1, PK * HID)
    bg2p = jnp.tile(bg2, PK).reshape(1, PK * HID)
    W4g2 = jnp.kron(jnp.eye(PK, dtype=jnp.float32), Wg2)
    batchp = jnp.concatenate(
        [batch, jnp.full((51200 - N,), NUM_GRAPHS, jnp.int32)])
    rr = jnp.arange(NPROWS, dtype=jnp.int32)
    node_k = (1024 * (rr // 256) + rr % 256)[:, None] \
        + 256 * jnp.arange(PK, dtype=jnp.int32)[None, :]
    batchq = batchp[node_k].T.reshape(PK, 4, BP3
                                      ).transpose(1, 0, 2).reshape(4, PK, 1, BP3)

    degp = _deg_kernel(srcdst, zeros32, ones32
                       ).reshape(2, NPROWS, PK * HID)
    h1p = _tc1_call(content_x.T, bert_x, profile_x.T, spacy_x.T,
                    Wc, b1, Wb, b2, Wp, b3, Ws, b4, Wg1)
    g1p = _scale_call(h1p, degp)
    acc1 = _edge_kernel(g1p.reshape(ACC_ROWS, HID), srcdst, zeros32
                        ).reshape(2, NPROWS, PK * HID)
    g2p = _tc2_call(acc1, g1p, degp, bg1p, W4g2)
    acc2 = _edge_kernel(g2p.reshape(ACC_ROWS, HID), srcdst, zeros32
                        ).reshape(2, NPROWS, PK * HID)
    return _tc3_call(acc2, g2p, degp, batchq, bg2p,
                     Wl1, bl1.reshape(1, HID), Wl2, bl2.reshape(1, OUT))
